# Initial kernel scaffold; baseline (speedup 1.0000x reference)
import jax
import jax.numpy as jnp
from jax.experimental import pallas as pl


def kernel(x, target, m_list):
    raise NotImplementedError("write your pallas kernel here")


# speedup vs baseline: 5.2021x; 5.2021x over previous
import jax
import jax.numpy as jnp
from jax import lax
from jax.experimental import pallas as pl
from jax.experimental.pallas import tpu as pltpu

_N = 16384
_C = 100
_S = 30.0
_BN = 1024
_NB = _N // _BN


def _body(x_ref, t_ref, ml_ref, out_ref):
    i = pl.program_id(0)
    x = x_ref[...]
    t = t_ref[...]
    ml = ml_ref[...]
    col = lax.broadcasted_iota(jnp.int32, (_BN, _C), 1)
    onehot = col == t
    logits = x * _S - jnp.where(onehot, ml * _S, 0.0)
    m = jnp.max(logits, axis=1, keepdims=True)
    se = jnp.sum(jnp.exp(logits - m), axis=1, keepdims=True)
    tgt = jnp.sum(jnp.where(onehot, logits, 0.0), axis=1, keepdims=True)
    part = jnp.sum(m + jnp.log(se) - tgt)

    @pl.when(i == 0)
    def _():
        out_ref[0, 0] = 0.0

    out_ref[0, 0] += part

    @pl.when(i == _NB - 1)
    def _():
        out_ref[0, 0] = out_ref[0, 0] / _N


def kernel(x, target, m_list):
    out = pl.pallas_call(
        _body,
        grid=(_NB,),
        in_specs=[
            pl.BlockSpec((_BN, _C), lambda i: (i, 0)),
            pl.BlockSpec((_BN, 1), lambda i: (i, 0)),
            pl.BlockSpec((1, _C), lambda i: (0, 0)),
        ],
        out_specs=pl.BlockSpec(memory_space=pltpu.SMEM),
        out_shape=jax.ShapeDtypeStruct((1, 1), jnp.float32),
        compiler_params=pltpu.CompilerParams(
            dimension_semantics=("arbitrary",),
        ),
    )(x, target.reshape(_N, 1), m_list.reshape(1, _C))
    return out[0, 0]


# speedup vs baseline: 5.9382x; 1.1415x over previous
import jax
import jax.numpy as jnp
from jax import lax
from jax.experimental import pallas as pl
from jax.experimental.pallas import tpu as pltpu

_N = 16384
_C = 100
_S = 30.0
_BN = 2048
_NB = _N // _BN


def _body(x_ref, t_ref, ml_ref, out_ref):
    i = pl.program_id(0)
    x = x_ref[...]
    t = t_ref[...]
    ml = ml_ref[...]
    col = lax.broadcasted_iota(jnp.int32, (_BN, _C), 1)
    onehot = col == t
    logits = x * _S - jnp.where(onehot, ml * _S, 0.0)
    m = jnp.max(logits, axis=1, keepdims=True)
    se = jnp.sum(jnp.exp(logits - m), axis=1, keepdims=True)
    tgt = jnp.sum(jnp.where(onehot, logits, 0.0), axis=1, keepdims=True)
    part = jnp.sum(m + jnp.log(se) - tgt)

    @pl.when(i == 0)
    def _():
        out_ref[0, 0] = 0.0

    out_ref[0, 0] += part

    @pl.when(i == _NB - 1)
    def _():
        out_ref[0, 0] = out_ref[0, 0] / _N


def kernel(x, target, m_list):
    out = pl.pallas_call(
        _body,
        grid=(_NB,),
        in_specs=[
            pl.BlockSpec((_BN, _C), lambda i: (i, 0)),
            pl.BlockSpec((_BN, 1), lambda i: (i, 0)),
            pl.BlockSpec((1, _C), lambda i: (0, 0)),
        ],
        out_specs=pl.BlockSpec(memory_space=pltpu.SMEM),
        out_shape=jax.ShapeDtypeStruct((1, 1), jnp.float32),
        compiler_params=pltpu.CompilerParams(
            dimension_semantics=("arbitrary",),
        ),
    )(x, target.reshape(_N, 1), m_list.reshape(1, _C))
    return out[0, 0]


# speedup vs baseline: 6.1648x; 1.0382x over previous
import jax
import jax.numpy as jnp
from jax import lax
from jax.experimental import pallas as pl
from jax.experimental.pallas import tpu as pltpu

_N = 16384
_C = 100
_S = 30.0
_BN = 4096
_NB = _N // _BN


def _body(x_ref, t_ref, ml_ref, out_ref):
    i = pl.program_id(0)
    x = x_ref[...]
    t = t_ref[...]
    ml = ml_ref[...]
    col = lax.broadcasted_iota(jnp.int32, (_BN, _C), 1)
    onehot = col == t
    logits = x * _S - jnp.where(onehot, ml * _S, 0.0)
    m = jnp.max(logits, axis=1, keepdims=True)
    se = jnp.sum(jnp.exp(logits - m), axis=1, keepdims=True)
    tgt = jnp.sum(jnp.where(onehot, logits, 0.0), axis=1, keepdims=True)
    part = jnp.sum(m + jnp.log(se) - tgt)

    @pl.when(i == 0)
    def _():
        out_ref[0, 0] = 0.0

    out_ref[0, 0] += part

    @pl.when(i == _NB - 1)
    def _():
        out_ref[0, 0] = out_ref[0, 0] / _N


def kernel(x, target, m_list):
    out = pl.pallas_call(
        _body,
        grid=(_NB,),
        in_specs=[
            pl.BlockSpec((_BN, _C), lambda i: (i, 0)),
            pl.BlockSpec((_BN, 1), lambda i: (i, 0)),
            pl.BlockSpec((1, _C), lambda i: (0, 0)),
        ],
        out_specs=pl.BlockSpec(memory_space=pltpu.SMEM),
        out_shape=jax.ShapeDtypeStruct((1, 1), jnp.float32),
        compiler_params=pltpu.CompilerParams(
            dimension_semantics=("arbitrary",),
        ),
    )(x, target.reshape(_N, 1), m_list.reshape(1, _C))
    return out[0, 0]
